# baseline (speedup 1.0000x reference)
import functools

import jax
import jax.numpy as jnp
from jax.experimental import pallas as pl
from jax.experimental.pallas import tpu as pltpu

B, IN_DIM, LATENT, K, OUT_DIM = 4096, 512, 128, 64, 512
ENC0, ENC1 = 1024, 512
DEC0, DEC1 = 512, 1024

ROWS = 1024


def _elu(x):
    em1 = jnp.tanh(x * 0.5) * (jnp.exp(x) + 1.0)
    return jnp.where(x > 0, x, em1)


def _fused_kernel(x_ref, w1_ref, b1_ref, w2_ref, b2_ref, mw_ref, mb_ref,
                  cb_ref, dw1_ref, db1_ref, dw2_ref, db2_ref, dw3_ref, db3_ref,
                  out_ref, q_ref, z_ref, tab_ref, c2_ref):
    cb = cb_ref[...]

    @pl.when(pl.program_id(0) == 0)
    def _build_table():
        d = _elu(jnp.dot(cb, dw1_ref[...], preferred_element_type=jnp.float32)
                 + db1_ref[...])
        d = _elu(jnp.dot(d, dw2_ref[...], preferred_element_type=jnp.float32)
                 + db2_ref[...])
        tab_ref[...] = (jnp.dot(d, dw3_ref[...],
                                preferred_element_type=jnp.float32)
                        + db3_ref[...])
        c2_ref[...] = jnp.sum(cb * cb, axis=1)[None, :]

    x = x_ref[...]
    h = _elu(jnp.dot(x, w1_ref[...], preferred_element_type=jnp.float32)
             + b1_ref[...])
    h = _elu(jnp.dot(h, w2_ref[...], preferred_element_type=jnp.float32)
             + b2_ref[...])
    z = jnp.dot(h, mw_ref[...], preferred_element_type=jnp.float32) + mb_ref[...]
    z_ref[...] = z

    z2 = jnp.sum(z * z, axis=-1, keepdims=True)
    zc = jax.lax.dot_general(
        z, cb, (((1,), (1,)), ((), ())), preferred_element_type=jnp.float32)
    scores = z2 + c2_ref[...] - 2.0 * zc
    mins = jnp.min(scores, axis=-1, keepdims=True)
    ks = jax.lax.broadcasted_iota(jnp.int32, scores.shape, 1)
    closest = jnp.min(jnp.where(scores == mins, ks, K), axis=-1)
    oh = (ks == closest[:, None]).astype(jnp.bfloat16)
    cb_hi = cb.astype(jnp.bfloat16)
    r1 = cb - cb_hi.astype(jnp.float32)
    cb_mid = r1.astype(jnp.bfloat16)
    cb_lo = (r1 - cb_mid.astype(jnp.float32)).astype(jnp.bfloat16)
    q_ref[...] = ((jnp.dot(oh, cb_hi, preferred_element_type=jnp.float32)
                   + jnp.dot(oh, cb_mid, preferred_element_type=jnp.float32))
                  + jnp.dot(oh, cb_lo, preferred_element_type=jnp.float32))
    out_ref[...] = jnp.dot(oh, tab_ref[...].astype(jnp.bfloat16),
                           preferred_element_type=jnp.float32)


@jax.jit
def kernel(input, enc_w1, enc_b1, enc_w2, enc_b2, mu_w, mu_b, dec_w1, dec_b1,
           dec_w2, dec_b2, dec_w3, dec_b3, codebook):
    grid = (B // ROWS,)
    const = lambda shape: pl.BlockSpec(shape, lambda i: (0, 0))
    input_hat, quantized, z = pl.pallas_call(
        _fused_kernel,
        grid=grid,
        in_specs=[
            pl.BlockSpec((ROWS, IN_DIM), lambda i: (i, 0)),
            const((IN_DIM, ENC0)),
            const((1, ENC0)),
            const((ENC0, ENC1)),
            const((1, ENC1)),
            const((ENC1, LATENT)),
            const((1, LATENT)),
            const((K, LATENT)),
            const((LATENT, DEC0)),
            const((1, DEC0)),
            const((DEC0, DEC1)),
            const((1, DEC1)),
            const((DEC1, OUT_DIM)),
            const((1, OUT_DIM)),
        ],
        out_specs=(
            pl.BlockSpec((ROWS, OUT_DIM), lambda i: (i, 0)),
            pl.BlockSpec((ROWS, LATENT), lambda i: (i, 0)),
            pl.BlockSpec((ROWS, LATENT), lambda i: (i, 0)),
        ),
        out_shape=(
            jax.ShapeDtypeStruct((B, OUT_DIM), jnp.float32),
            jax.ShapeDtypeStruct((B, LATENT), jnp.float32),
            jax.ShapeDtypeStruct((B, LATENT), jnp.float32),
        ),
        scratch_shapes=[
            pltpu.VMEM((K, OUT_DIM), jnp.float32),
            pltpu.VMEM((1, K), jnp.float32),
        ],
    )(input, enc_w1, enc_b1.reshape(1, ENC0), enc_w2, enc_b2.reshape(1, ENC1),
      mu_w, mu_b.reshape(1, LATENT), codebook, dec_w1,
      dec_b1.reshape(1, DEC0), dec_w2, dec_b2.reshape(1, DEC1), dec_w3,
      dec_b3.reshape(1, OUT_DIM))
    return (input_hat, quantized, z)


# speedup vs baseline: 1.1194x; 1.1194x over previous
import functools

import jax
import jax.numpy as jnp
from jax.experimental import pallas as pl
from jax.experimental.pallas import tpu as pltpu

B, IN_DIM, LATENT, K, OUT_DIM = 4096, 512, 128, 64, 512
ENC0, ENC1 = 1024, 512
DEC0, DEC1 = 512, 1024

ROWS = 1024


def _elu(x):
    em1 = jnp.tanh(x * 0.5) * (jnp.exp(x) + 1.0)
    return jnp.where(x > 0, x, em1)


def _fused_kernel(x_ref, w1_ref, b1_ref, w2_ref, b2_ref, mw_ref, mb_ref,
                  cb_ref, dw1_ref, db1_ref, dw2_ref, db2_ref, dw3_ref, db3_ref,
                  out_ref, q_ref, z_ref, tab_ref, c2_ref):
    cb = cb_ref[...]

    @pl.when(pl.program_id(0) == 0)
    def _build_table():
        d = _elu(jnp.dot(cb, dw1_ref[...], preferred_element_type=jnp.float32)
                 + db1_ref[...])
        d = _elu(jnp.dot(d, dw2_ref[...], preferred_element_type=jnp.float32)
                 + db2_ref[...])
        tab_ref[...] = (jnp.dot(d, dw3_ref[...],
                                preferred_element_type=jnp.float32)
                        + db3_ref[...])
        c2_ref[...] = jnp.sum(cb * cb, axis=1)[None, :]

    x = x_ref[...]
    h = _elu(jnp.dot(x, w1_ref[...], preferred_element_type=jnp.float32)
             + b1_ref[...])
    h = _elu(jnp.dot(h, w2_ref[...], preferred_element_type=jnp.float32)
             + b2_ref[...])
    z = jnp.dot(h, mw_ref[...], preferred_element_type=jnp.float32) + mb_ref[...]
    z_ref[...] = z

    z2 = jnp.sum(z * z, axis=-1, keepdims=True)
    zc = jax.lax.dot_general(
        z, cb, (((1,), (1,)), ((), ())), preferred_element_type=jnp.float32)
    scores = z2 + c2_ref[...] - 2.0 * zc
    ks = jax.lax.broadcasted_iota(jnp.int32, scores.shape, 1)
    closest = jnp.argmin(scores, axis=-1).astype(jnp.int32)
    oh = (ks == closest[:, None]).astype(jnp.float32)
    q_ref[...] = jnp.dot(oh, cb, preferred_element_type=jnp.float32)
    out_ref[...] = jnp.dot(oh.astype(jnp.bfloat16),
                           tab_ref[...].astype(jnp.bfloat16),
                           preferred_element_type=jnp.float32)


@jax.jit
def kernel(input, enc_w1, enc_b1, enc_w2, enc_b2, mu_w, mu_b, dec_w1, dec_b1,
           dec_w2, dec_b2, dec_w3, dec_b3, codebook):
    grid = (B // ROWS,)
    const = lambda shape: pl.BlockSpec(shape, lambda i: (0, 0))
    input_hat, quantized, z = pl.pallas_call(
        _fused_kernel,
        grid=grid,
        in_specs=[
            pl.BlockSpec((ROWS, IN_DIM), lambda i: (i, 0)),
            const((IN_DIM, ENC0)),
            const((1, ENC0)),
            const((ENC0, ENC1)),
            const((1, ENC1)),
            const((ENC1, LATENT)),
            const((1, LATENT)),
            const((K, LATENT)),
            const((LATENT, DEC0)),
            const((1, DEC0)),
            const((DEC0, DEC1)),
            const((1, DEC1)),
            const((DEC1, OUT_DIM)),
            const((1, OUT_DIM)),
        ],
        out_specs=(
            pl.BlockSpec((ROWS, OUT_DIM), lambda i: (i, 0)),
            pl.BlockSpec((ROWS, LATENT), lambda i: (i, 0)),
            pl.BlockSpec((ROWS, LATENT), lambda i: (i, 0)),
        ),
        out_shape=(
            jax.ShapeDtypeStruct((B, OUT_DIM), jnp.float32),
            jax.ShapeDtypeStruct((B, LATENT), jnp.float32),
            jax.ShapeDtypeStruct((B, LATENT), jnp.float32),
        ),
        scratch_shapes=[
            pltpu.VMEM((K, OUT_DIM), jnp.float32),
            pltpu.VMEM((1, K), jnp.float32),
        ],
    )(input, enc_w1, enc_b1.reshape(1, ENC0), enc_w2, enc_b2.reshape(1, ENC1),
      mu_w, mu_b.reshape(1, LATENT), codebook, dec_w1,
      dec_b1.reshape(1, DEC0), dec_w2, dec_b2.reshape(1, DEC1), dec_w3,
      dec_b3.reshape(1, OUT_DIM))
    return (input_hat, quantized, z)
